# Initial kernel scaffold; baseline (speedup 1.0000x reference)
import jax
import jax.numpy as jnp
from jax.experimental import pallas as pl


def kernel(tokens, embedding_weight, fusion_embedding_weight):
    raise NotImplementedError("write your pallas kernel here")


# speedup vs baseline: 2.6132x; 2.6132x over previous
import functools

import jax
import jax.numpy as jnp
from jax import lax
from jax.experimental import pallas as pl
from jax.experimental.pallas import tpu as pltpu
from jax.experimental.pallas import tpu_sc as plsc

NUM_WORKERS = 32
LANES = 16
IDX_BLK = 128
CHUNK = 512


def kernel(tokens, embedding_weight, fusion_embedding_weight):
    B, S = tokens.shape
    V, D = embedding_weight.shape
    F = fusion_embedding_weight.shape[0]
    N = B * S
    n_per_w = N // NUM_WORKERS
    n_chunks = n_per_w // CHUNK
    blk_per_chunk = CHUNK // IDX_BLK
    grp_per_blk = IDX_BLK // LANES

    tok2d = tokens.reshape(N // IDX_BLK, IDX_BLK)
    mesh = plsc.VectorSubcoreMesh(core_axis_name="c", subcore_axis_name="s")

    @functools.partial(
        pl.kernel,
        mesh=mesh,
        out_type=jax.ShapeDtypeStruct((N, D), jnp.float32),
        compiler_params=pltpu.CompilerParams(
            needs_layout_passes=False, use_tc_tiling_on_sc=False
        ),
        scratch_types=[
            pltpu.VMEM((F, D), jnp.float32),
            pltpu.VMEM((blk_per_chunk, IDX_BLK), jnp.int32),
            pltpu.VMEM((blk_per_chunk, IDX_BLK), jnp.int32),
            pltpu.VMEM((CHUNK, D), jnp.float32),
            pltpu.SemaphoreType.DMA,
        ],
    )
    def run(tok_hbm, emb_hbm, fus_hbm, out_hbm, fus_v, tok_v, idx_v, rows_v, sem):
        wid = lax.axis_index("s") * 2 + lax.axis_index("c")
        row_base_w = wid * (n_per_w // IDX_BLK)

        pltpu.sync_copy(fus_hbm, fus_v)

        def chunk_body(g, carry):
            row_base = row_base_w + g * blk_per_chunk
            tok_base = (row_base_w + g * blk_per_chunk) * IDX_BLK
            pltpu.sync_copy(tok_hbm.at[pl.ds(row_base, blk_per_chunk)], tok_v)

            for r in range(blk_per_chunk):
                for c in range(grp_per_blk):
                    t = tok_v[r, pl.ds(c * LANES, LANES)]
                    idx_v[r, pl.ds(c * LANES, LANES)] = jnp.where(t < V, t, 0)

            copies = [
                pltpu.make_async_copy(
                    emb_hbm.at[idx_v.at[r]],
                    rows_v.at[pl.ds(r * IDX_BLK, IDX_BLK)],
                    sem,
                )
                for r in range(blk_per_chunk)
            ]
            for cp in copies:
                cp.start()
            for cp in copies:
                cp.wait()

            for r in range(blk_per_chunk):
                for c in range(grp_per_blk):
                    t = tok_v[r, pl.ds(c * LANES, LANES)]
                    fm = t >= V
                    cnt = plsc.all_reduce_population_count(fm)

                    @pl.when(cnt[0] > 0)
                    def _():
                        fidx = jnp.where(fm, t - V, 0)
                        rowpos = lax.iota(jnp.int32, LANES) + (r * IDX_BLK + c * LANES)

                        def col(j, carry):
                            colv = jnp.full((LANES,), 0, jnp.int32) + j
                            vals = plsc.load_gather(fus_v, [fidx, colv])
                            plsc.store_scatter(rows_v, [rowpos, colv], vals, mask=fm)
                            return carry

                        lax.fori_loop(0, D, col, 0)

            pltpu.sync_copy(rows_v, out_hbm.at[pl.ds(tok_base, CHUNK)])
            return carry

        lax.fori_loop(0, n_chunks, chunk_body, 0)

    out = run(tok2d, embedding_weight, fusion_embedding_weight)
    return out.reshape(B, S, D)


# speedup vs baseline: 2.7050x; 1.0351x over previous
import functools

import jax
import jax.numpy as jnp
from jax import lax
from jax.experimental import pallas as pl
from jax.experimental.pallas import tpu as pltpu
from jax.experimental.pallas import tpu_sc as plsc

NUM_WORKERS = 32
LANES = 16
IDX_BLK = 128
CHUNK = 256
NSLOT = 2


def kernel(tokens, embedding_weight, fusion_embedding_weight):
    B, S = tokens.shape
    V, D = embedding_weight.shape
    F = fusion_embedding_weight.shape[0]
    N = B * S
    n_per_w = N // NUM_WORKERS
    n_chunks = n_per_w // CHUNK
    blk = CHUNK // IDX_BLK
    grp_per_blk = IDX_BLK // LANES

    tok2d = tokens.reshape(N // IDX_BLK, IDX_BLK)
    mesh = plsc.VectorSubcoreMesh(core_axis_name="c", subcore_axis_name="s")

    @functools.partial(
        pl.kernel,
        mesh=mesh,
        out_type=jax.ShapeDtypeStruct((N, D), jnp.float32),
        compiler_params=pltpu.CompilerParams(
            needs_layout_passes=False, use_tc_tiling_on_sc=False
        ),
        scratch_types=[
            pltpu.VMEM((F, D), jnp.float32),
            [pltpu.VMEM((blk, IDX_BLK), jnp.int32)] * NSLOT,
            [pltpu.VMEM((blk, IDX_BLK), jnp.int32)] * NSLOT,
            [pltpu.VMEM((CHUNK, D), jnp.float32)] * NSLOT,
            [pltpu.SemaphoreType.DMA] * NSLOT,
            [pltpu.SemaphoreType.DMA] * NSLOT,
            [pltpu.SemaphoreType.DMA] * NSLOT,
        ],
    )
    def run(tok_hbm, emb_hbm, fus_hbm, out_hbm, fus_v, tok_v, idx_v, rows_v,
            sem_t, sem_g, sem_o):
        wid = lax.axis_index("s") * 2 + lax.axis_index("c")
        row_base_w = wid * (n_per_w // IDX_BLK)

        pltpu.sync_copy(fus_hbm, fus_v)

        def tok_copy(g, b):
            return pltpu.make_async_copy(
                tok_hbm.at[pl.ds(row_base_w + g * blk, blk)], tok_v[b], sem_t[b]
            )

        def out_copy(g, b):
            return pltpu.make_async_copy(
                rows_v[b],
                out_hbm.at[pl.ds((row_base_w + g * blk) * IDX_BLK, CHUNK)],
                sem_o[b],
            )

        def gather_copies(b):
            return [
                pltpu.make_async_copy(
                    emb_hbm.at[idx_v[b].at[r]],
                    rows_v[b].at[pl.ds(r * IDX_BLK, IDX_BLK)],
                    sem_g[b],
                )
                for r in range(blk)
            ]

        for b in range(NSLOT):
            tok_copy(b, b).start()

        def step(i, carry):
            for b in range(NSLOT):
                g = i * NSLOT + b
                tok_copy(g, b).wait()

                for r in range(blk):
                    for c in range(grp_per_blk):
                        t = tok_v[b][r, pl.ds(c * LANES, LANES)]
                        idx_v[b][r, pl.ds(c * LANES, LANES)] = jnp.where(t < V, t, 0)

                @pl.when(g >= NSLOT)
                def _():
                    out_copy(g - NSLOT, b).wait()

                for cp in gather_copies(b):
                    cp.start()
                for cp in gather_copies(b):
                    cp.wait()

                for r in range(blk):
                    for c in range(grp_per_blk):
                        t = tok_v[b][r, pl.ds(c * LANES, LANES)]
                        fm = t >= V
                        cnt = plsc.all_reduce_population_count(fm)

                        @pl.when(cnt[0] > 0)
                        def _():
                            fidx = jnp.where(fm, t - V, 0)
                            rowpos = lax.iota(jnp.int32, LANES) + (
                                r * IDX_BLK + c * LANES
                            )

                            def col(j, cc):
                                colv = jnp.full((LANES,), 0, jnp.int32) + j
                                vals = plsc.load_gather(fus_v, [fidx, colv])
                                plsc.store_scatter(
                                    rows_v[b], [rowpos, colv], vals, mask=fm
                                )
                                return cc

                            lax.fori_loop(0, D, col, 0)

                @pl.when(g + NSLOT < n_chunks)
                def _():
                    tok_copy(g + NSLOT, b).start()

                out_copy(g, b).start()
            return carry

        lax.fori_loop(0, n_chunks // NSLOT, step, 0)

        for b in range(NSLOT):
            out_copy(n_chunks - NSLOT + b, b).wait()

    out = run(tok2d, embedding_weight, fusion_embedding_weight)
    return out.reshape(B, S, D)
